# baseline (speedup 1.0000x reference)
import functools

import jax
import jax.numpy as jnp
from jax import lax
from jax.experimental import pallas as pl
from jax.experimental.pallas import tpu as pltpu
from jax.experimental.pallas import tpu_sc as plsc

_D = 8
_LANES = 16


def _sc_codebook(x_planes, n_rows):
    info = plsc.get_sparse_core_info()
    nw = info.num_cores * info.num_subcores
    rows_per_w = n_rows // nw
    mesh = plsc.VectorSubcoreMesh(core_axis_name="c", subcore_axis_name="s")

    @functools.partial(
        pl.kernel,
        out_type=jax.ShapeDtypeStruct((n_rows,), jnp.int32),
        mesh=mesh,
        scratch_types=[
            pltpu.VMEM((rows_per_w * _D,), jnp.float32),
            pltpu.VMEM((rows_per_w,), jnp.int32),
        ],
        compiler_params=pltpu.CompilerParams(needs_layout_passes=False),
    )
    def k(x_hbm, out_hbm, xbuf, obuf):
        wid = lax.axis_index("s") * info.num_cores + lax.axis_index("c")
        base = wid * rows_per_w
        pltpu.sync_copy(x_hbm.at[pl.ds(base * _D, rows_per_w * _D)], xbuf)

        def body(i, carry):
            goff = (i >> 3) * 512 + (i & 7) * _LANES
            off = (i >> 3) * 128 + (i & 7) * _LANES
            acc = jnp.zeros((_LANES,), jnp.int32)
            tacc = jnp.zeros((_LANES,), jnp.int32)
            cols = []
            for j in range(_D):
                c, p = j // 4, j % 4
                col = xbuf[pl.ds(goff + (c * (rows_per_w * 4) + p * 128), _LANES)]
                cols.append(col)
                m = col > 0.501953125
                acc = acc | jnp.where(m, jnp.int32(1 << j), jnp.int32(0))
                u = plsc.bitcast(col, jnp.uint32)
                tied = (u - jnp.uint32(0x3EFF8000)) <= jnp.uint32(0x10000)
                tacc = tacc | jnp.where(tied, jnp.int32(1 << j), jnp.int32(0))
            obuf[pl.ds(off, _LANES)] = acc

            @pl.when(jnp.any(tacc != 0))
            def _tie_fix():
                g0 = jnp.zeros((_LANES,), jnp.float32)
                sq = []
                for j in range(_D):
                    col = cols[j]
                    u = plsc.bitcast(col, jnp.uint32)
                    rnd = (u + jnp.uint32(0x7FFF)) + ((u >> 16) & jnp.uint32(1))
                    xb = plsc.bitcast(rnd & jnp.uint32(0xFFFF0000), jnp.float32)
                    g0 = g0 + jnp.where(xb > 0.5, xb, jnp.float32(0.0))
                    sq.append(col * col)
                y = [sq[s] + sq[s + 4] for s in range(4)]
                z = [y[s] + y[s + 2] for s in range(2)]
                s2 = z[0] + z[1]
                v = (acc & 0x55) + ((acc >> 1) & 0x55)
                v = (v & 0x33) + ((v >> 2) & 0x33)
                v = (v + (v >> 4)) & 0x0F
                n0 = v.astype(jnp.float32)
                tg = 2.0 * g0
                d0 = (s2 - tg) + n0
                d1 = (s2 - (tg + 1.0)) + (n0 + 1.0)
                fixed = acc | jnp.where(d1 < d0, tacc, jnp.int32(0))
                obuf[pl.ds(off, _LANES)] = fixed

            return carry

        lax.fori_loop(0, rows_per_w // _LANES, body, 0)
        pltpu.sync_copy(obuf, out_hbm.at[pl.ds(base, rows_per_w)])

    return k(x_planes)


def kernel(projection_windows, emb_weight):
    shape = projection_windows.shape
    b, t = shape[0], shape[1]
    n_rows = b * t
    planes = jnp.transpose(
        projection_windows.reshape(b, t // 128, 128, 2, 4),
        (0, 3, 1, 4, 2)).reshape(-1)
    out = _sc_codebook(planes, n_rows)
    return out.reshape(shape[:-2])


# speedup vs baseline: 1.0600x; 1.0600x over previous
import functools

import jax
import jax.numpy as jnp
from jax import lax
from jax.experimental import pallas as pl
from jax.experimental.pallas import tpu as pltpu
from jax.experimental.pallas import tpu_sc as plsc

_D = 8
_LANES = 16
_NCH = 8


def _sc_codebook(x_planes, n_rows):
    info = plsc.get_sparse_core_info()
    nw = info.num_cores * info.num_subcores
    rows_per_w = n_rows // nw
    half = rows_per_w * 4
    ch_words = half // _NCH
    ch_groups = rows_per_w // _NCH // _LANES
    mesh = plsc.VectorSubcoreMesh(core_axis_name="c", subcore_axis_name="s")

    @functools.partial(
        pl.kernel,
        out_type=jax.ShapeDtypeStruct((n_rows,), jnp.int32),
        mesh=mesh,
        scratch_types=[
            pltpu.VMEM((2, 2 * ch_words), jnp.float32),
            pltpu.VMEM((rows_per_w,), jnp.int32),
            pltpu.SemaphoreType.DMA,
            pltpu.SemaphoreType.DMA,
        ],
        compiler_params=pltpu.CompilerParams(needs_layout_passes=False),
    )
    def k(x_hbm, out_hbm, xbuf, obuf, sem0, sem1):
        wid = lax.axis_index("s") * info.num_cores + lax.axis_index("c")
        base = wid * rows_per_w
        base_in = base * _D
        sems = (sem0, sem1)

        def start(ch):
            b = ch % 2
            return (
                pltpu.async_copy(
                    x_hbm.at[pl.ds(base_in + ch * ch_words, ch_words)],
                    xbuf.at[b, pl.ds(0, ch_words)], sems[b]),
                pltpu.async_copy(
                    x_hbm.at[pl.ds(base_in + half + ch * ch_words, ch_words)],
                    xbuf.at[b, pl.ds(ch_words, ch_words)], sems[b]),
            )

        pending = start(0)
        for ch in range(_NCH):
            for h in pending:
                h.wait()
            if ch + 1 < _NCH:
                pending = start(ch + 1)
            b = ch % 2
            out_base = ch * (rows_per_w // _NCH)

            def body(i, carry):
                goff = (i >> 3) * 512 + (i & 7) * _LANES
                off = out_base + (i >> 3) * 128 + (i & 7) * _LANES
                acc = jnp.zeros((_LANES,), jnp.int32)
                tacc = jnp.zeros((_LANES,), jnp.int32)
                g0 = jnp.zeros((_LANES,), jnp.float32)
                sq = []
                for j in range(_D):
                    c, p = j // 4, j % 4
                    col = xbuf[b, pl.ds(goff + (c * ch_words + p * 128), _LANES)]
                    u = plsc.bitcast(col, jnp.uint32)
                    rnd = (u + jnp.uint32(0x7FFF)) + ((u >> 16) & jnp.uint32(1))
                    xb = plsc.bitcast(rnd & jnp.uint32(0xFFFF0000), jnp.float32)
                    m = xb > 0.5
                    acc = acc | jnp.where(m, jnp.int32(1 << j), jnp.int32(0))
                    tacc = tacc | jnp.where(
                        xb == 0.5, jnp.int32(1 << j), jnp.int32(0))
                    g0 = g0 + jnp.where(m, xb, jnp.float32(0.0))
                    sq.append(col * col)
                y = [sq[s] + sq[s + 4] for s in range(4)]
                z = [y[s] + y[s + 2] for s in range(2)]
                s2 = z[0] + z[1]
                v = (acc & 0x55) + ((acc >> 1) & 0x55)
                v = (v & 0x33) + ((v >> 2) & 0x33)
                v = (v + (v >> 4)) & 0x0F
                n0 = v.astype(jnp.float32)
                tg = 2.0 * g0
                d0 = (s2 - tg) + n0
                d1 = (s2 - (tg + 1.0)) + (n0 + 1.0)
                acc = acc | jnp.where(d1 < d0, tacc, jnp.int32(0))
                obuf[pl.ds(off, _LANES)] = acc
                return carry

            lax.fori_loop(0, ch_groups, body, 0)

        pltpu.sync_copy(obuf, out_hbm.at[pl.ds(base, rows_per_w)])

    return k(x_planes)


def kernel(projection_windows, emb_weight):
    shape = projection_windows.shape
    b, t = shape[0], shape[1]
    n_rows = b * t
    planes = jnp.transpose(
        projection_windows.reshape(b, t // 128, 128, 2, 4),
        (0, 3, 1, 4, 2)).reshape(-1)
    out = _sc_codebook(planes, n_rows)
    return out.reshape(shape[:-2])


# speedup vs baseline: 1.1855x; 1.1184x over previous
import functools

import jax
import jax.numpy as jnp
from jax import lax
from jax.experimental import pallas as pl
from jax.experimental.pallas import tpu as pltpu
from jax.experimental.pallas import tpu_sc as plsc

_D = 8
_LANES = 16


def _sc_codebook(x_planes, n_rows):
    info = plsc.get_sparse_core_info()
    nw = info.num_cores * info.num_subcores
    rows_per_w = n_rows // nw
    t_blks = rows_per_w // 128
    mesh = plsc.VectorSubcoreMesh(core_axis_name="c", subcore_axis_name="s")

    @functools.partial(
        pl.kernel,
        out_type=jax.ShapeDtypeStruct((nw // 8 * t_blks, 8, 128), jnp.int32),
        mesh=mesh,
        scratch_types=[
            pltpu.VMEM((rows_per_w * _D,), jnp.float32),
            pltpu.VMEM((t_blks, 1, 128), jnp.int32),
        ],
        compiler_params=pltpu.CompilerParams(needs_layout_passes=False),
    )
    def k(x_hbm, out_hbm, xbuf, obuf):
        wid = lax.axis_index("s") * info.num_cores + lax.axis_index("c")
        base_in = wid * rows_per_w * _D
        pltpu.sync_copy(x_hbm.at[pl.ds(base_in, rows_per_w * _D)], xbuf)

        def body(i, carry):
            goff = (i >> 3) * 512 + (i & 7) * _LANES
            acc = jnp.zeros((_LANES,), jnp.int32)
            tacc = jnp.zeros((_LANES,), jnp.int32)
            g0 = jnp.zeros((_LANES,), jnp.float32)
            sq = []
            for j in range(_D):
                c, p = j // 4, j % 4
                col = xbuf[pl.ds(goff + (c * (rows_per_w * 4) + p * 128), _LANES)]
                u = plsc.bitcast(col, jnp.uint32)
                rnd = (u + jnp.uint32(0x7FFF)) + ((u >> 16) & jnp.uint32(1))
                xb = plsc.bitcast(rnd & jnp.uint32(0xFFFF0000), jnp.float32)
                m = xb > 0.5
                acc = acc | jnp.where(m, jnp.int32(1 << j), jnp.int32(0))
                tacc = tacc | jnp.where(
                    xb == 0.5, jnp.int32(1 << j), jnp.int32(0))
                g0 = g0 + jnp.where(m, xb, jnp.float32(0.0))
                sq.append(col * col)
            y = [sq[s] + sq[s + 4] for s in range(4)]
            z = [y[s] + y[s + 2] for s in range(2)]
            s2 = z[0] + z[1]
            v = (acc & 0x55) + ((acc >> 1) & 0x55)
            v = (v & 0x33) + ((v >> 2) & 0x33)
            v = (v + (v >> 4)) & 0x0F
            n0 = v.astype(jnp.float32)
            tg = 2.0 * g0
            d0 = (s2 - tg) + n0
            d1 = (s2 - (tg + 1.0)) + (n0 + 1.0)
            acc = acc | jnp.where(d1 < d0, tacc, jnp.int32(0))
            obuf[i >> 3, 0, pl.ds((i & 7) * _LANES, _LANES)] = acc
            return carry

        lax.fori_loop(0, rows_per_w // _LANES, body, 0)
        pltpu.sync_copy(
            obuf,
            out_hbm.at[pl.ds((wid // 8) * t_blks, t_blks),
                       pl.ds(wid % 8, 1), :])

    return k(x_planes)


def kernel(projection_windows, emb_weight):
    shape = projection_windows.shape
    b, t = shape[0], shape[1]
    n_rows = b * t
    planes = jnp.transpose(
        projection_windows.reshape(b, t // 128, 128, 2, 4),
        (0, 3, 1, 4, 2)).reshape(-1)
    out = _sc_codebook(planes, n_rows)
    return (out.reshape(b // 8, t // 128, 8, 128)
            .transpose(0, 2, 1, 3).reshape(b, t))
